# baseline (speedup 1.0000x reference)
import functools

import jax
import jax.numpy as jnp
from jax import lax
from jax.experimental import pallas as pl
from jax.experimental.pallas import tpu as pltpu
from jax.experimental.pallas import tpu_sc as plsc

N_IN = 64
N_OUT = 16
SEQ = 4096
DM = 128

L = 16
NC = 2
NS = 16
NW = NC * NS

CHUNKS = 256
CW = SEQ * DM // CHUNKS
OUT_ROWS = N_OUT * CHUNKS
PER_TILE = OUT_ROWS // NW
BATCH = 16
NBATCH = PER_TILE // BATCH


def _top16(w_v, g_v):
  def merge(ka, va, kb, vb):
    kb_r = lax.rev(kb, (0,))
    vb_r = lax.rev(vb, (0,))
    take_a = ka >= kb_r
    km = jnp.where(take_a, ka, kb_r)
    vm = jnp.where(take_a, va, vb_r)
    return plsc.sort_key_val(km, vm, descending=True)

  ks, vs = [], []
  for i in range(N_IN // L):
    s = w_v[pl.ds(i * L, L)] + g_v[pl.ds(i * L, L)]
    idx = lax.iota(jnp.int32, L) + i * L
    k, v = plsc.sort_key_val(s, idx, descending=True)
    ks.append(k)
    vs.append(v)
  k01, v01 = merge(ks[0], vs[0], ks[1], vs[1])
  k23, v23 = merge(ks[2], vs[2], ks[3], vs[3])
  _, top = merge(k01, v01, k23, v23)
  return top


def _body(states_hbm, w_hbm, g_hbm, out_hbm,
          w_v, g_v, topidx_v, idx_v, buf_v, gsem, ssem):
  wid = lax.axis_index("s") * NC + lax.axis_index("c")

  pltpu.sync_copy(w_hbm, w_v)
  pltpu.sync_copy(g_hbm, g_v)
  topidx_v[...] = _top16(w_v, g_v)

  base = wid * PER_TILE
  lanes = lax.iota(jnp.int32, L)

  def start_gather(b):
    g_vec = base + b * BATCH + lanes
    p_vec = lax.shift_right_logical(g_vec, 8)
    j_vec = lax.bitwise_and(g_vec, CHUNKS - 1)
    src = plsc.load_gather(topidx_v, [p_vec]) * CHUNKS + j_vec
    idx_v[b % 2, :] = src
    return pltpu.async_copy(
        states_hbm.at[idx_v.at[b % 2]], buf_v.at[b % 2], gsem)

  gathers = [None] * NBATCH
  scatters = [None] * NBATCH
  gathers[0] = start_gather(0)
  for b in range(NBATCH):
    gathers[b].wait()
    scatters[b] = pltpu.async_copy(
        buf_v.at[b % 2], out_hbm.at[pl.ds(base + b * BATCH, BATCH)], ssem)
    if b + 1 < NBATCH:
      if b >= 1:
        scatters[b - 1].wait()
      gathers[b + 1] = start_gather(b + 1)
  scatters[NBATCH - 2].wait()
  scatters[NBATCH - 1].wait()


@jax.jit
def _merge_filter(states_flat, w_merge, gumbel):
  mesh = plsc.VectorSubcoreMesh(core_axis_name="c", subcore_axis_name="s")
  run = functools.partial(
      pl.kernel,
      out_type=jax.ShapeDtypeStruct((OUT_ROWS, CW), jnp.float32),
      mesh=mesh,
      scratch_types=[
          pltpu.VMEM((N_IN,), jnp.float32),
          pltpu.VMEM((N_IN,), jnp.float32),
          pltpu.VMEM((L,), jnp.int32),
          pltpu.VMEM((2, L), jnp.int32),
          pltpu.VMEM((2, BATCH, CW), jnp.float32),
          pltpu.SemaphoreType.DMA,
          pltpu.SemaphoreType.DMA,
      ],
      compiler_params=pltpu.CompilerParams(needs_layout_passes=False),
  )(_body)
  return run(states_flat, w_merge, gumbel)


def kernel(states, w_merge):
  gumbel = jax.random.gumbel(jax.random.key(42), (N_IN,), jnp.float32)
  states_flat = states.reshape(N_IN * CHUNKS, CW)
  out = _merge_filter(states_flat, w_merge, gumbel)
  return out.reshape(N_OUT, SEQ, DM)
